# baseline (speedup 1.0000x reference)
import functools

import jax
import jax.numpy as jnp
from jax import lax
from jax.experimental import pallas as pl
from jax.experimental.pallas import tpu as pltpu
from jax.experimental.pallas import tpu_sc as plsc

B, L, H, FF, E, C = 2, 4096, 1024, 4096, 8, 2
K = (L * C) // E
P = B * E
ROWS = P * K
ONE_F32_BITS = 0x3F800001


def _router_body(x_ref, wg_ref, s_ref):
    xb = x_ref[0]
    logits = jnp.dot(xb, wg_ref[...], preferred_element_type=jnp.float32)
    m = jnp.max(logits, axis=-1, keepdims=True)
    ex = jnp.exp(logits - m)
    s_ref[0] = ex / jnp.sum(ex, axis=-1, keepdims=True)


def _router(x, wg):
    return pl.pallas_call(
        _router_body,
        grid=(B, L // 512),
        in_specs=[
            pl.BlockSpec((1, 512, H), lambda b, i: (b, i, 0)),
            pl.BlockSpec((H, E), lambda b, i: (0, 0)),
        ],
        out_specs=pl.BlockSpec((1, 512, E), lambda b, i: (b, i, 0)),
        out_shape=jax.ShapeDtypeStruct((B, L, E), jnp.float32),
    )(x, wg)


def _threshold_body(st_ref, t_ref):
    s = st_ref[...]
    lo = jnp.zeros((P, 1), jnp.int32)
    hi = jnp.full((P, 1), ONE_F32_BITS, jnp.int32)

    def step(_, carry):
        lo, hi = carry
        mid = lo + (hi - lo) // 2
        t = lax.bitcast_convert_type(mid, jnp.float32)
        cnt = jnp.sum((s >= t).astype(jnp.int32), axis=-1, keepdims=True)
        ok = cnt >= K
        return jnp.where(ok, mid, lo), jnp.where(ok, hi, mid)

    lo, hi = lax.fori_loop(0, 31, step, (lo, hi))
    t_ref[...] = lax.bitcast_convert_type(lo, jnp.float32)


def _threshold(st):
    return pl.pallas_call(
        _threshold_body,
        out_shape=jax.ShapeDtypeStruct((P, 1), jnp.float32),
    )(st)


def _compact_body(st_hbm, t_hbm, il_hbm, ig_hbm, g_hbm,
                  s_v, t_v, il_v, ig_v, g_v):
    c = lax.axis_index("c")
    s = lax.axis_index("s")

    @pl.when(s < E)
    def _():
        p = c * E + s
        base = c * L
        pltpu.sync_copy(st_hbm.at[p], s_v)
        pltpu.sync_copy(t_hbm.at[p], t_v)
        tsplat = t_v[...]

        def phase(sel_fn, limit, cnt0):
            def step(i, cnt):
                sv = s_v[pl.ds(i * 16, 16)]
                idxv = (i * 16 + lax.iota(jnp.int32, 16)).astype(jnp.int32)
                m = sel_fn(sv)
                pos = cnt + jnp.cumsum(m.astype(jnp.int32)) - 1
                keep = m & (pos < limit)
                posc = jnp.where(keep, pos, 0)
                plsc.store_scatter(il_v, [posc], idxv, mask=keep)
                plsc.store_scatter(ig_v, [posc], idxv + base, mask=keep)
                plsc.store_scatter(g_v, [posc], sv, mask=keep)
                return cnt + plsc.all_reduce_population_count(m)
            return lax.fori_loop(0, L // 16, step, cnt0)

        n1 = phase(lambda sv: sv > tsplat, K, jnp.zeros((16,), jnp.int32))
        phase(lambda sv: sv == tsplat, K, n1)

        pltpu.sync_copy(il_v, il_hbm.at[p])
        pltpu.sync_copy(ig_v, ig_hbm.at[p])
        pltpu.sync_copy(g_v, g_hbm.at[p])


def _compact(st, tb):
    mesh = plsc.VectorSubcoreMesh(core_axis_name="c", subcore_axis_name="s")
    f = functools.partial(
        pl.kernel, mesh=mesh,
        compiler_params=pltpu.CompilerParams(needs_layout_passes=False),
        out_type=[
            jax.ShapeDtypeStruct((P, K), jnp.int32),
            jax.ShapeDtypeStruct((P, K), jnp.int32),
            jax.ShapeDtypeStruct((P, K), jnp.float32),
        ],
        scratch_types=[
            pltpu.VMEM((L,), jnp.float32),
            pltpu.VMEM((16,), jnp.float32),
            pltpu.VMEM((K,), jnp.int32),
            pltpu.VMEM((K,), jnp.int32),
            pltpu.VMEM((K,), jnp.float32),
        ],
    )(_compact_body)
    return f(st, tb)


_GCH = 32


def _gather_body(x_hbm, ig_hbm, xe_hbm, idx_v, rows_v, sem):
    c = lax.axis_index("c")
    s = lax.axis_index("s")
    wid = c * 16 + s
    nch = (ROWS // 32) // _GCH
    pltpu.sync_copy(ig_hbm.at[pl.ds(wid * nch, nch)], idx_v)
    for j in range(nch):
        pltpu.async_copy(x_hbm.at[idx_v.at[j]], rows_v, sem).wait()
        pltpu.sync_copy(rows_v, xe_hbm.at[pl.ds(wid * nch * _GCH + j * _GCH, _GCH)])


def _gather(x2, ig):
    nch = (ROWS // 32) // _GCH
    mesh = plsc.VectorSubcoreMesh(core_axis_name="c", subcore_axis_name="s")
    f = functools.partial(
        pl.kernel, mesh=mesh,
        compiler_params=pltpu.CompilerParams(needs_layout_passes=False),
        out_type=jax.ShapeDtypeStruct((ROWS, H), jnp.float32),
        scratch_types=[
            pltpu.VMEM((nch, _GCH), jnp.int32),
            pltpu.VMEM((_GCH, H), jnp.float32),
            pltpu.SemaphoreType.DMA,
        ],
    )(_gather_body)
    return f(x2, ig.reshape(ROWS // _GCH, _GCH))


def _mlp_body(xe_ref, w1_ref, b1_ref, w2_ref, b2_ref, g_ref, o_ref):
    f = pl.program_id(1)
    h = jnp.dot(xe_ref[...].astype(jnp.bfloat16), w1_ref[0],
                preferred_element_type=jnp.float32)
    h = h + b1_ref[0]
    h = 0.5 * h * (1.0 + lax.erf(h * 0.7071067811865476))
    t2 = jnp.dot(h.astype(jnp.bfloat16), w2_ref[0],
                 preferred_element_type=jnp.float32)
    g = g_ref[...]

    @pl.when(f == 0)
    def _():
        o_ref[...] = g * (t2 + b2_ref[0])

    @pl.when(f != 0)
    def _():
        o_ref[...] += g * t2


def _mlp(xe, w1, b1, w2, b2, g):
    fblk = 512
    return pl.pallas_call(
        _mlp_body,
        grid=(P, FF // fblk),
        in_specs=[
            pl.BlockSpec((K, H), lambda p, f: (p, 0)),
            pl.BlockSpec((1, H, fblk), lambda p, f: (p % E, 0, f)),
            pl.BlockSpec((1, 1, fblk), lambda p, f: (p % E, 0, f)),
            pl.BlockSpec((1, fblk, H), lambda p, f: (p % E, f, 0)),
            pl.BlockSpec((1, 1, H), lambda p, f: (p % E, 0, 0)),
            pl.BlockSpec((K, 1), lambda p, f: (p, 0)),
        ],
        out_specs=pl.BlockSpec((K, H), lambda p, f: (p, 0)),
        out_shape=jax.ShapeDtypeStruct((ROWS, H), jnp.float32),
    )(xe, w1.astype(jnp.bfloat16), b1.reshape(E, 1, FF),
      w2.astype(jnp.bfloat16), b2.reshape(E, 1, H), g)


def _scatter_body(pred_hbm, ig_hbm, z_hbm, out_hbm, idx_v, prow_v, obuf_v):
    c = lax.axis_index("c")
    s = lax.axis_index("s")
    wid = c * 16 + s
    pltpu.sync_copy(z_hbm, out_hbm.at[pl.ds(wid * 256, 256)])
    plsc.subcore_barrier()
    for e in range(E):
        p = c * E + e
        for hh in range(2):
            ih = e * 2 + hh
            pltpu.sync_copy(ig_hbm.at[p * 32 + s * 2 + hh], idx_v.at[ih])
            pltpu.sync_copy(pred_hbm.at[pl.ds(p * K + s * 64 + hh * 32, 32)],
                            prow_v)
            pltpu.sync_copy(out_hbm.at[idx_v.at[ih]], obuf_v)

            def addrow(r, _):
                for j in range(H // 16):
                    plsc.addupdate(obuf_v.at[r, pl.ds(j * 16, 16)],
                                   prow_v[r, pl.ds(j * 16, 16)])
                return 0
            lax.fori_loop(0, 32, addrow, 0)
            pltpu.sync_copy(obuf_v, out_hbm.at[idx_v.at[ih]])
        plsc.subcore_barrier()


def _scatter(pred, ig, zeros):
    mesh = plsc.VectorSubcoreMesh(core_axis_name="c", subcore_axis_name="s")
    f = functools.partial(
        pl.kernel, mesh=mesh,
        compiler_params=pltpu.CompilerParams(needs_layout_passes=False),
        out_type=jax.ShapeDtypeStruct((B * L, H), jnp.float32),
        scratch_types=[
            pltpu.VMEM((2 * E, 32), jnp.int32),
            pltpu.VMEM((32, H), jnp.float32),
            pltpu.VMEM((32, H), jnp.float32),
        ],
    )(_scatter_body)
    return f(pred, ig.reshape(ROWS // 32, 32), zeros)


def kernel(x, Wg, W1, b1, W2, b2):
    S = _router(x, Wg)
    st = jnp.transpose(S, (0, 2, 1)).reshape(P, L)
    t = _threshold(st)
    tb = jnp.broadcast_to(t, (P, 16))
    il, ig, g = _compact(st, tb)
    xe = _gather(x.reshape(B * L, H), ig)
    pred = _mlp(xe, W1, b1, W2, b2, g.reshape(ROWS, 1))
    zeros = jnp.zeros((256, H), jnp.float32)
    return _scatter(pred, ig, zeros).reshape(B, L, H)


# speedup vs baseline: 2.1291x; 2.1291x over previous
import functools

import jax
import jax.numpy as jnp
from jax import lax
from jax.experimental import pallas as pl
from jax.experimental.pallas import tpu as pltpu
from jax.experimental.pallas import tpu_sc as plsc

B, L, H, FF, E, C = 2, 4096, 1024, 4096, 8, 2
K = (L * C) // E
P = B * E
ROWS = P * K
ONE_F32_BITS = 0x3F800001


def _router_body(x_ref, wg_ref, s_ref):
    xb = x_ref[0]
    logits = jnp.dot(xb, wg_ref[...], preferred_element_type=jnp.float32)
    m = jnp.max(logits, axis=-1, keepdims=True)
    ex = jnp.exp(logits - m)
    s_ref[0] = ex / jnp.sum(ex, axis=-1, keepdims=True)


def _router(x, wg):
    return pl.pallas_call(
        _router_body,
        grid=(B, L // 512),
        in_specs=[
            pl.BlockSpec((1, 512, H), lambda b, i: (b, i, 0)),
            pl.BlockSpec((H, E), lambda b, i: (0, 0)),
        ],
        out_specs=pl.BlockSpec((1, 512, E), lambda b, i: (b, i, 0)),
        out_shape=jax.ShapeDtypeStruct((B, L, E), jnp.float32),
    )(x, wg)


def _threshold_body(st_ref, t_ref):
    s = st_ref[...]
    lo = jnp.zeros((P, 1), jnp.int32)
    hi = jnp.full((P, 1), ONE_F32_BITS, jnp.int32)

    def step(_, carry):
        lo, hi = carry
        mid = lo + (hi - lo) // 2
        t = lax.bitcast_convert_type(mid, jnp.float32)
        cnt = jnp.sum((s >= t).astype(jnp.int32), axis=-1, keepdims=True)
        ok = cnt >= K
        return jnp.where(ok, mid, lo), jnp.where(ok, hi, mid)

    lo, hi = lax.fori_loop(0, 31, step, (lo, hi))
    t_ref[...] = lax.bitcast_convert_type(lo, jnp.float32)


def _threshold(st):
    return pl.pallas_call(
        _threshold_body,
        out_shape=jax.ShapeDtypeStruct((P, 1), jnp.float32),
    )(st)


def _compact_body(st_hbm, t_hbm, il_hbm, ig_hbm, g_hbm,
                  s_v, t_v, il_v, ig_v, g_v):
    c = lax.axis_index("c")
    s = lax.axis_index("s")

    @pl.when(s < E)
    def _():
        p = c * E + s
        base = c * L
        pltpu.sync_copy(st_hbm.at[p], s_v)
        pltpu.sync_copy(t_hbm.at[p], t_v)
        tsplat = t_v[...]

        def phase(sel_fn, limit, cnt0):
            def step(i, cnt):
                sv = s_v[pl.ds(i * 16, 16)]
                idxv = (i * 16 + lax.iota(jnp.int32, 16)).astype(jnp.int32)
                m = sel_fn(sv)
                pos = cnt + jnp.cumsum(m.astype(jnp.int32)) - 1
                keep = m & (pos < limit)
                posc = jnp.where(keep, pos, 0)
                plsc.store_scatter(il_v, [posc], idxv, mask=keep)
                plsc.store_scatter(ig_v, [posc], idxv + base, mask=keep)
                plsc.store_scatter(g_v, [posc], sv, mask=keep)
                return cnt + plsc.all_reduce_population_count(m)
            return lax.fori_loop(0, L // 16, step, cnt0)

        n1 = phase(lambda sv: sv > tsplat, K, jnp.zeros((16,), jnp.int32))
        phase(lambda sv: sv == tsplat, K, n1)

        pltpu.sync_copy(il_v, il_hbm.at[p])
        pltpu.sync_copy(ig_v, ig_hbm.at[p])
        pltpu.sync_copy(g_v, g_hbm.at[p])


def _compact(st, tb):
    mesh = plsc.VectorSubcoreMesh(core_axis_name="c", subcore_axis_name="s")
    f = functools.partial(
        pl.kernel, mesh=mesh,
        compiler_params=pltpu.CompilerParams(needs_layout_passes=False),
        out_type=[
            jax.ShapeDtypeStruct((P, K), jnp.int32),
            jax.ShapeDtypeStruct((P, K), jnp.int32),
            jax.ShapeDtypeStruct((P, K), jnp.float32),
        ],
        scratch_types=[
            pltpu.VMEM((L,), jnp.float32),
            pltpu.VMEM((16,), jnp.float32),
            pltpu.VMEM((K,), jnp.int32),
            pltpu.VMEM((K,), jnp.int32),
            pltpu.VMEM((K,), jnp.float32),
        ],
    )(_compact_body)
    return f(st, tb)


_GCH = 32


def _gather_body(x_hbm, ig_hbm, xe_hbm, idx_v, rows_v, sem):
    c = lax.axis_index("c")
    s = lax.axis_index("s")
    wid = c * 16 + s
    nch = (ROWS // 32) // _GCH
    pltpu.sync_copy(ig_hbm.at[pl.ds(wid * nch, nch)], idx_v)
    for j in range(nch):
        pltpu.async_copy(x_hbm.at[idx_v.at[j]], rows_v, sem).wait()
        pltpu.sync_copy(rows_v, xe_hbm.at[pl.ds(wid * nch * _GCH + j * _GCH, _GCH)])


def _gather(x2, ig):
    nch = (ROWS // 32) // _GCH
    mesh = plsc.VectorSubcoreMesh(core_axis_name="c", subcore_axis_name="s")
    f = functools.partial(
        pl.kernel, mesh=mesh,
        compiler_params=pltpu.CompilerParams(needs_layout_passes=False),
        out_type=jax.ShapeDtypeStruct((ROWS, H), jnp.float32),
        scratch_types=[
            pltpu.VMEM((nch, _GCH), jnp.int32),
            pltpu.VMEM((_GCH, H), jnp.float32),
            pltpu.SemaphoreType.DMA,
        ],
    )(_gather_body)
    return f(x2, ig.reshape(ROWS // _GCH, _GCH))


def _mlp_body(xe_ref, w1_ref, b1_ref, w2_ref, b2_ref, g_ref, o_ref):
    f = pl.program_id(1)
    h = jnp.dot(xe_ref[...].astype(jnp.bfloat16), w1_ref[0],
                preferred_element_type=jnp.float32)
    h = h + b1_ref[0]
    h = 0.5 * h * (1.0 + lax.erf(h * 0.7071067811865476))
    t2 = jnp.dot(h.astype(jnp.bfloat16), w2_ref[0],
                 preferred_element_type=jnp.float32)
    g = g_ref[...]

    @pl.when(f == 0)
    def _():
        o_ref[...] = g * (t2 + b2_ref[0])

    @pl.when(f != 0)
    def _():
        o_ref[...] += g * t2


def _mlp(xe, w1, b1, w2, b2, g):
    fblk = 512
    return pl.pallas_call(
        _mlp_body,
        grid=(P, FF // fblk),
        in_specs=[
            pl.BlockSpec((K, H), lambda p, f: (p, 0)),
            pl.BlockSpec((1, H, fblk), lambda p, f: (p % E, 0, f)),
            pl.BlockSpec((1, 1, fblk), lambda p, f: (p % E, 0, f)),
            pl.BlockSpec((1, fblk, H), lambda p, f: (p % E, f, 0)),
            pl.BlockSpec((1, 1, H), lambda p, f: (p % E, 0, 0)),
            pl.BlockSpec((K, 1), lambda p, f: (p, 0)),
        ],
        out_specs=pl.BlockSpec((K, H), lambda p, f: (p, 0)),
        out_shape=jax.ShapeDtypeStruct((ROWS, H), jnp.float32),
    )(xe, w1.astype(jnp.bfloat16), b1.reshape(E, 1, FF),
      w2.astype(jnp.bfloat16), b2.reshape(E, 1, H), g)


def _scatter_body(pred_hbm, ig_hbm, z_hbm, out_hbm, idx_v, prow_v, obuf_v):
    c = lax.axis_index("c")
    s = lax.axis_index("s")
    wid = c * 16 + s
    def zrow(r, _):
        for j in range(H // 16):
            obuf_v[r, pl.ds(j * 16, 16)] = jnp.zeros((16,), jnp.float32)
        return 0
    lax.fori_loop(0, 32, zrow, 0)
    for w in range(8):
        pltpu.sync_copy(obuf_v, out_hbm.at[pl.ds(wid * 256 + w * 32, 32)])
    plsc.subcore_barrier()
    for e in range(E):
        p = c * E + e
        for hh in range(2):
            ih = e * 2 + hh
            pltpu.sync_copy(ig_hbm.at[p * 32 + s * 2 + hh], idx_v.at[ih])
            pltpu.sync_copy(pred_hbm.at[pl.ds(p * K + s * 64 + hh * 32, 32)],
                            prow_v)
            pltpu.sync_copy(out_hbm.at[idx_v.at[ih]], obuf_v)

            def addrow(r, _):
                for j in range(H // 16):
                    plsc.addupdate(obuf_v.at[r, pl.ds(j * 16, 16)],
                                   prow_v[r, pl.ds(j * 16, 16)])
                return 0
            lax.fori_loop(0, 32, addrow, 0)
            pltpu.sync_copy(obuf_v, out_hbm.at[idx_v.at[ih]])
        plsc.subcore_barrier()


def _scatter(pred, ig, zeros):
    mesh = plsc.VectorSubcoreMesh(core_axis_name="c", subcore_axis_name="s")
    f = functools.partial(
        pl.kernel, mesh=mesh,
        compiler_params=pltpu.CompilerParams(needs_layout_passes=False),
        out_type=jax.ShapeDtypeStruct((B * L, H), jnp.float32),
        scratch_types=[
            pltpu.VMEM((2 * E, 32), jnp.int32),
            pltpu.VMEM((32, H), jnp.float32),
            pltpu.VMEM((32, H), jnp.float32),
        ],
    )(_scatter_body)
    return f(pred, ig.reshape(ROWS // 32, 32), zeros)


def kernel(x, Wg, W1, b1, W2, b2):
    S = _router(x, Wg)
    st = jnp.transpose(S, (0, 2, 1)).reshape(P, L)
    t = _threshold(st)
    tb = jnp.broadcast_to(t, (P, 16))
    il, ig, g = _compact(st, tb)
    xe = _gather(x.reshape(B * L, H), ig)
    pred = _mlp(xe, W1, b1, W2, b2, g.reshape(ROWS, 1))
    zeros = jnp.zeros((256, H), jnp.float32)
    return _scatter(pred, ig, zeros).reshape(B, L, H)
